# Initial kernel scaffold; baseline (speedup 1.0000x reference)
import jax
import jax.numpy as jnp
from jax.experimental import pallas as pl


def kernel(x1, edge_index1, batch1, x2, edge_index2, batch2, context, espf1, mixfp1, espf2, mixfp2, cnv, pw, params):
    raise NotImplementedError("write your pallas kernel here")


# speedup vs baseline: 1.0701x; 1.0701x over previous
import functools
import math

import jax
import jax.numpy as jnp
from jax import lax
from jax.experimental import pallas as pl
from jax.experimental.pallas import tpu as pltpu


def _mm(a, w, bias=None, act=None, ln=None, bm=1024):
    M, K = a.shape
    K2, N = w.shape
    assert K == K2
    bm = min(bm, M)
    assert M % bm == 0
    grid = (M // bm,)

    def body(*refs):
        a_ref, w_ref = refs[0], refs[1]
        o_ref = refs[-1]
        idx = 2
        z = jnp.dot(a_ref[:], w_ref[:], preferred_element_type=jnp.float32)
        if bias is not None:
            z = z + refs[idx][:].reshape(1, N)
            idx += 1
        if act is not None:
            z = act(z)
        if ln is not None:
            g_ref, b_ref = refs[idx], refs[idx + 1]
            mu = jnp.mean(z, axis=-1, keepdims=True)
            var = jnp.mean((z - mu) ** 2, axis=-1, keepdims=True)
            z = (z - mu) * lax.rsqrt(var + 1e-5) * g_ref[:].reshape(1, N) \
                + b_ref[:].reshape(1, N)
        o_ref[:] = z

    in_specs = [
        pl.BlockSpec((bm, K), lambda i: (i, 0)),
        pl.BlockSpec((K, N), lambda i: (0, 0)),
    ]
    args = [a, w]
    if bias is not None:
        in_specs.append(pl.BlockSpec((N,), lambda i: (0,)))
        args.append(bias)
    if ln is not None:
        in_specs.append(pl.BlockSpec((N,), lambda i: (0,)))
        in_specs.append(pl.BlockSpec((N,), lambda i: (0,)))
        args.extend(ln)
    return pl.pallas_call(
        body,
        grid=grid,
        in_specs=in_specs,
        out_specs=pl.BlockSpec((bm, N), lambda i: (i, 0)),
        out_shape=jax.ShapeDtypeStruct((M, N), jnp.float32),
    )(*args)


def _relu(z):
    return jnp.maximum(z, 0.0)


def _elu(z):
    return jnp.where(z > 0, z, jnp.exp(jnp.minimum(z, 0.0)) - 1.0)


def _gat_pre(x, W, A_s, A_d, want_h, bm):
    M, K = x.shape
    HC = W.shape[1]
    H = A_s.shape[1]
    assert M % bm == 0
    grid = (M // bm,)

    def body(x_ref, w_ref, as_ref, ad_ref, *outs):
        h = jnp.dot(x_ref[:], w_ref[:], preferred_element_type=jnp.float32)
        s1 = jnp.dot(h, as_ref[:], preferred_element_type=jnp.float32)
        s2 = jnp.dot(h, ad_ref[:], preferred_element_type=jnp.float32)
        if want_h:
            outs[0][:] = h
            outs[1][:] = s1
            outs[2][:] = s2
        else:
            outs[0][:] = s1
            outs[1][:] = s2

    out_shapes = []
    out_specs = []
    if want_h:
        out_shapes.append(jax.ShapeDtypeStruct((M, HC), jnp.float32))
        out_specs.append(pl.BlockSpec((bm, HC), lambda i: (i, 0)))
    out_shapes.append(jax.ShapeDtypeStruct((M, H), jnp.float32))
    out_specs.append(pl.BlockSpec((bm, H), lambda i: (i, 0)))
    out_shapes.append(jax.ShapeDtypeStruct((M, H), jnp.float32))
    out_specs.append(pl.BlockSpec((bm, H), lambda i: (i, 0)))

    res = pl.pallas_call(
        body,
        grid=grid,
        in_specs=[
            pl.BlockSpec((bm, K), lambda i: (i, 0)),
            pl.BlockSpec((K, HC), lambda i: (0, 0)),
            pl.BlockSpec((HC, H), lambda i: (0, 0)),
            pl.BlockSpec((HC, H), lambda i: (0, 0)),
        ],
        out_specs=out_specs,
        out_shape=out_shapes,
    )(x, W, A_s, A_d)
    if want_h:
        return res[0], res[1], res[2]
    return None, res[0], res[1]


def _head_diag(a):
    H, C = a.shape
    m = jnp.zeros((H * C, H), jnp.float32)
    return m.at[jnp.arange(H * C), jnp.arange(H * C) // C].set(a.reshape(-1))


def _gat_postA(z3, Wp, bias, bm=1000):
    H, n, Fp = z3.shape
    HC = Wp.shape[1]
    C = HC // H
    assert n % bm == 0
    grid = (H, n // bm)

    def body(z_ref, w_ref, b_ref, o_ref):
        zb = z_ref[0]
        den = zb[:, 78:79]
        o = jnp.dot(zb, w_ref[:], preferred_element_type=jnp.float32)
        o = o / jnp.maximum(den, 1e-16) + b_ref[0]
        o_ref[:] = _elu(o)

    return pl.pallas_call(
        body,
        grid=grid,
        in_specs=[
            pl.BlockSpec((1, bm, Fp), lambda h, m: (h, m, 0)),
            pl.BlockSpec((Fp, C), lambda h, m: (0, h)),
            pl.BlockSpec((1, 1, C), lambda h, m: (h, 0, 0)),
        ],
        out_specs=pl.BlockSpec((bm, C), lambda h, m: (m, h)),
        out_shape=jax.ShapeDtypeStruct((n, HC), jnp.float32),
    )(z3, Wp, bias.reshape(H, 1, C))


def _gat_postB(zB, bias, bm=1000):
    n, Fp = zB.shape
    C = 128
    grid = (n // bm,)

    def body(z_ref, b_ref, o_ref):
        zb = z_ref[:]
        den = zb[:, C:C + 1]
        o = zb[:, :C] / jnp.maximum(den, 1e-16) + b_ref[:].reshape(1, C)
        o_ref[:] = _elu(o)

    return pl.pallas_call(
        body,
        grid=grid,
        in_specs=[
            pl.BlockSpec((bm, Fp), lambda i: (i, 0)),
            pl.BlockSpec((C,), lambda i: (0,)),
        ],
        out_specs=pl.BlockSpec((bm, C), lambda i: (i, 0)),
        out_shape=jax.ShapeDtypeStruct((n, C), jnp.float32),
    )(zB, bias)


def _gmp(x, batch, nb):
    n, C = x.shape
    bm = 1000
    grid = (n // bm,)

    def body(b_ref, x_ref, o_ref):
        i = pl.program_id(0)

        @pl.when(i == 0)
        def _init():
            o_ref[:] = jnp.full((nb, C), -jnp.inf, jnp.float32)

        def step(r, _):
            b = b_ref[i * bm + r]
            row = x_ref[pl.ds(r, 1), :]
            o_ref[pl.ds(b, 1), :] = jnp.maximum(o_ref[pl.ds(b, 1), :], row)
            return 0

        lax.fori_loop(0, bm, step, 0)

        @pl.when(i == grid[0] - 1)
        def _fix():
            v = o_ref[:]
            o_ref[:] = jnp.where(v > -jnp.inf, v, 0.0)

    return pl.pallas_call(
        body,
        grid_spec=pltpu.PrefetchScalarGridSpec(
            num_scalar_prefetch=1,
            grid=grid,
            in_specs=[pl.BlockSpec((bm, C), lambda i, b: (i, 0))],
            out_specs=pl.BlockSpec((nb, C), lambda i, b: (0, 0)),
        ),
        out_shape=jax.ShapeDtypeStruct((nb, C), jnp.float32),
    )(batch, x)


def _edges_with_loops(ei, n):
    loops = jnp.arange(n, dtype=ei.dtype)
    return jnp.concatenate([ei, jnp.stack([loops, loops])], axis=1)


def _edge_aggA(x, ei, s1, s2):
    n = x.shape[0]
    src, dst = ei[0], ei[1]
    e = s1[src] + s2[dst]
    ex = jnp.exp(jnp.where(e > 0, e, 0.2 * e))
    xt = jnp.concatenate(
        [x, jnp.ones((n, 1), jnp.float32), jnp.zeros((n, 1), jnp.float32)], 1)
    data = xt[src][:, None, :] * ex[:, :, None]
    z = jax.ops.segment_sum(data, dst, num_segments=n)
    return jnp.transpose(z, (1, 0, 2))


def _edge_aggB(h, ei, s1, s2):
    n = h.shape[0]
    src, dst = ei[0], ei[1]
    e = s1[src, 0] + s2[dst, 0]
    ex = jnp.exp(jnp.where(e > 0, e, 0.2 * e))
    ht = jnp.concatenate(
        [h, jnp.ones((n, 1), jnp.float32), jnp.zeros((n, 7), jnp.float32)], 1)
    return jax.ops.segment_sum(ht[src] * ex[:, None], dst, num_segments=n)


def _graph_branch(x, ei, batch, nb, P, pfx, fcW, fcB):
    n = x.shape[0]
    ei = _edges_with_loops(ei, n)

    W = P[pfx + 'aW']
    A_s = _head_diag(P[pfx + 'aAs'])
    A_d = _head_diag(P[pfx + 'aAd'])
    _, s1, s2 = _gat_pre(x, W, A_s, A_d, want_h=False, bm=1000)
    z3 = _edge_aggA(x, ei, s1, s2)
    Wp = jnp.concatenate([W, jnp.zeros((2, W.shape[1]), jnp.float32)], 0)
    h1 = _gat_postA(z3, Wp, P[pfx + 'aB'])

    W2 = P[pfx + 'bW']
    A2s = _head_diag(P[pfx + 'bAs'])
    A2d = _head_diag(P[pfx + 'bAd'])
    hB, t1, t2 = _gat_pre(h1, W2, A2s, A2d, want_h=True, bm=1000)
    zB = _edge_aggB(hB, ei, t1, t2)
    h2 = _gat_postB(zB, P[pfx + 'bB'])

    pooled = _gmp(h2, batch, nb)
    return _mm(pooled, fcW, bias=fcB, act=_relu)


def _mha_tail(allf, P):
    nb = allf.shape[0]
    a2 = allf.reshape(nb * 9, 256)
    q = _mm(a2, P['WQ'], bm=1024).reshape(nb, 9, 4, 64)
    k = _mm(a2, P['WK'], bm=1024).reshape(nb, 9, 4, 64)
    v = _mm(a2, P['WV'], bm=1024).reshape(nb, 9, 4, 64)
    r0 = _mm(a2, P['WR'], bm=1024).reshape(nb, 9, 256)
    att = jax.nn.softmax(
        jnp.einsum('bqhd,bkhd->bhqk', q, k) / (64.0 ** 0.5), axis=-1)
    r = jnp.einsum('bhqk,bkhd->bqhd', att, v).reshape(nb, 9, 256)
    mh = _relu(r + r0).reshape(nb, 2304)

    t = _ln_op(mh, P['lnG'], P['lnB'])
    sqrt2 = math.sqrt(2.0)
    t = _mm(t, P['t1W'], bias=P['t1B'],
            act=lambda z: 0.5 * z * (1.0 + lax.erf(z / sqrt2)))
    t = _mm(t, P['t2W'], bias=P['t2B'])
    return t


def _ln_op(x, g, b):
    M, N = x.shape

    def body(x_ref, g_ref, b_ref, o_ref):
        z = x_ref[:]
        mu = jnp.mean(z, axis=-1, keepdims=True)
        var = jnp.mean((z - mu) ** 2, axis=-1, keepdims=True)
        o_ref[:] = (z - mu) * lax.rsqrt(var + 1e-5) * g_ref[:].reshape(1, N) \
            + b_ref[:].reshape(1, N)

    return pl.pallas_call(
        body,
        grid=(1,),
        in_specs=[
            pl.BlockSpec((M, N), lambda i: (0, 0)),
            pl.BlockSpec((N,), lambda i: (0,)),
            pl.BlockSpec((N,), lambda i: (0,)),
        ],
        out_specs=pl.BlockSpec((M, N), lambda i: (0, 0)),
        out_shape=jax.ShapeDtypeStruct((M, N), jnp.float32),
    )(x, g, b)


def kernel(x1, edge_index1, batch1, x2, edge_index2, batch2, context,
           espf1, mixfp1, espf2, mixfp2, cnv, pw, params):
    P = params
    nb = context.shape[0]

    s1 = _graph_branch(x1, edge_index1, batch1, nb, P, 'g1', P['fc1W'], P['fc1B'])
    s2 = _graph_branch(x2, edge_index2, batch2, nb, P, 'g2', P['fc2W'], P['fc2B'])

    ctx = _mm(context[:, 0, :], P['pcW'], bias=P['pcB'], act=_relu,
              ln=(P['pcG'], P['pcBe']))
    cn = _mm(cnv[:, 0, :], P['pnW'], bias=P['pnB'], act=_relu,
             ln=(P['pnG'], P['pnBe']))
    y = _mm(pw[:, 0, :], P['pwW1'], bias=P['pwB1'], act=_relu,
            ln=(P['pwG1'], P['pwBe1']))
    pwf = _mm(y, P['pwW2'], bias=P['pwB2'], act=_relu,
              ln=(P['pwG2'], P['pwBe2']))
    e1 = _mm(espf1, P['pe1W'], bias=P['pe1B'], act=_relu, ln=(P['pe1G'], P['pe1Be']))
    e2 = _mm(espf2, P['pe2W'], bias=P['pe2B'], act=_relu, ln=(P['pe2G'], P['pe2Be']))
    m1 = _mm(mixfp1, P['pm1W'], bias=P['pm1B'], act=_relu, ln=(P['pm1G'], P['pm1Be']))
    m2 = _mm(mixfp2, P['pm2W'], bias=P['pm2B'], act=_relu, ln=(P['pm2G'], P['pm2Be']))

    allf = jnp.stack([s1, m1, e1, ctx, cn, pwf, s2, m2, e2], axis=1)
    mx = _mha_tail(allf, P) + allf.reshape(nb, 2304)

    af = mx.reshape(nb, 9, 256)
    d1 = af[:, :3, :].reshape(nb, 768)
    d2 = af[:, 6:, :].reshape(nb, 768)
    cell = af[:, 3:6, :].reshape(nb, 768)
    comb = _mm(jnp.concatenate([d1, d2], axis=1), P['combW'])
    ther = comb * cell
    out = _mm(_ln_op(ther, P['trG'], P['trB']),
              jnp.pad(P['tW'], ((0, 0), (0, 127))),
              bias=jnp.pad(P['tB'], (0, 127)))[:, :1]
    return (out, ther)


# speedup vs baseline: 1.1849x; 1.1072x over previous
import functools
import math

import jax
import jax.numpy as jnp
from jax import lax
from jax.experimental import pallas as pl
from jax.experimental.pallas import tpu as pltpu
from jax.experimental.pallas import tpu_sc as plsc


def _mm(a, w, bias=None, act=None, ln=None, bm=1024):
    M, K = a.shape
    K2, N = w.shape
    assert K == K2
    bm = min(bm, M)
    assert M % bm == 0
    grid = (M // bm,)

    def body(*refs):
        a_ref, w_ref = refs[0], refs[1]
        o_ref = refs[-1]
        idx = 2
        z = jnp.dot(a_ref[:], w_ref[:], preferred_element_type=jnp.float32)
        if bias is not None:
            z = z + refs[idx][:].reshape(1, N)
            idx += 1
        if act is not None:
            z = act(z)
        if ln is not None:
            g_ref, b_ref = refs[idx], refs[idx + 1]
            mu = jnp.mean(z, axis=-1, keepdims=True)
            var = jnp.mean((z - mu) ** 2, axis=-1, keepdims=True)
            z = (z - mu) * lax.rsqrt(var + 1e-5) * g_ref[:].reshape(1, N) \
                + b_ref[:].reshape(1, N)
        o_ref[:] = z

    in_specs = [
        pl.BlockSpec((bm, K), lambda i: (i, 0)),
        pl.BlockSpec((K, N), lambda i: (0, 0)),
    ]
    args = [a, w]
    if bias is not None:
        in_specs.append(pl.BlockSpec((N,), lambda i: (0,)))
        args.append(bias)
    if ln is not None:
        in_specs.append(pl.BlockSpec((N,), lambda i: (0,)))
        in_specs.append(pl.BlockSpec((N,), lambda i: (0,)))
        args.extend(ln)
    return pl.pallas_call(
        body,
        grid=grid,
        in_specs=in_specs,
        out_specs=pl.BlockSpec((bm, N), lambda i: (i, 0)),
        out_shape=jax.ShapeDtypeStruct((M, N), jnp.float32),
    )(*args)


def _relu(z):
    return jnp.maximum(z, 0.0)


def _elu(z):
    return jnp.where(z > 0, z, jnp.exp(jnp.minimum(z, 0.0)) - 1.0)


def _gat_pre(x, W, A_s, A_d, want_h, bm):
    M, K = x.shape
    HC = W.shape[1]
    H = A_s.shape[1]
    assert M % bm == 0
    grid = (M // bm,)

    def body(x_ref, w_ref, as_ref, ad_ref, *outs):
        h = jnp.dot(x_ref[:], w_ref[:], preferred_element_type=jnp.float32)
        s1 = jnp.dot(h, as_ref[:], preferred_element_type=jnp.float32)
        s2 = jnp.dot(h, ad_ref[:], preferred_element_type=jnp.float32)
        if want_h:
            outs[0][:] = h
            outs[1][:] = s1
            outs[2][:] = s2
        else:
            outs[0][:] = s1
            outs[1][:] = s2

    out_shapes = []
    out_specs = []
    if want_h:
        out_shapes.append(jax.ShapeDtypeStruct((M, HC), jnp.float32))
        out_specs.append(pl.BlockSpec((bm, HC), lambda i: (i, 0)))
    out_shapes.append(jax.ShapeDtypeStruct((M, H), jnp.float32))
    out_specs.append(pl.BlockSpec((bm, H), lambda i: (i, 0)))
    out_shapes.append(jax.ShapeDtypeStruct((M, H), jnp.float32))
    out_specs.append(pl.BlockSpec((bm, H), lambda i: (i, 0)))

    res = pl.pallas_call(
        body,
        grid=grid,
        in_specs=[
            pl.BlockSpec((bm, K), lambda i: (i, 0)),
            pl.BlockSpec((K, HC), lambda i: (0, 0)),
            pl.BlockSpec((HC, H), lambda i: (0, 0)),
            pl.BlockSpec((HC, H), lambda i: (0, 0)),
        ],
        out_specs=out_specs,
        out_shape=out_shapes,
    )(x, W, A_s, A_d)
    if want_h:
        return res[0], res[1], res[2]
    return None, res[0], res[1]


def _head_diag(a):
    H, C = a.shape
    m = jnp.zeros((H * C, H), jnp.float32)
    return m.at[jnp.arange(H * C), jnp.arange(H * C) // C].set(a.reshape(-1))


def _gat_postA(z4, Wp, bias, n, bm=1000):
    _, H, n_pad, Fp = z4.shape
    HC = Wp.shape[1]
    C = HC // H
    assert n % bm == 0
    grid = (H, n // bm)

    def body(z_ref, w_ref, b_ref, o_ref):
        zb = z_ref[0, 0] + z_ref[1, 0]
        den = zb[:, 78:79]
        o = jnp.dot(zb, w_ref[:], preferred_element_type=jnp.float32)
        o = o / jnp.maximum(den, 1e-16) + b_ref[0]
        o_ref[:] = _elu(o)

    return pl.pallas_call(
        body,
        grid=grid,
        in_specs=[
            pl.BlockSpec((2, 1, bm, Fp), lambda h, m: (0, h, m, 0)),
            pl.BlockSpec((Fp, C), lambda h, m: (0, h)),
            pl.BlockSpec((1, 1, C), lambda h, m: (h, 0, 0)),
        ],
        out_specs=pl.BlockSpec((bm, C), lambda h, m: (m, h)),
        out_shape=jax.ShapeDtypeStruct((n, HC), jnp.float32),
    )(z4, Wp, bias.reshape(H, 1, C))


def _gat_postB(zB, bias, n, bm=1000):
    _, n_pad, Fp = zB.shape
    C = 128
    grid = (n // bm,)

    def body(z_ref, b_ref, o_ref):
        zb = z_ref[0] + z_ref[1]
        den = zb[:, C:C + 1]
        o = zb[:, :C] / jnp.maximum(den, 1e-16) + b_ref[:].reshape(1, C)
        o_ref[:] = _elu(o)

    return pl.pallas_call(
        body,
        grid=grid,
        in_specs=[
            pl.BlockSpec((2, bm, Fp), lambda i: (0, i, 0)),
            pl.BlockSpec((C,), lambda i: (0,)),
        ],
        out_specs=pl.BlockSpec((bm, C), lambda i: (i, 0)),
        out_shape=jax.ShapeDtypeStruct((n, C), jnp.float32),
    )(zB, bias)


def _gmp(x, batch, nb):
    n, C = x.shape
    bm = 1000
    grid = (n // bm,)

    def body(b_ref, x_ref, o_ref):
        i = pl.program_id(0)

        @pl.when(i == 0)
        def _init():
            o_ref[:] = jnp.full((nb, C), -jnp.inf, jnp.float32)

        def step(r, _):
            b = b_ref[i * bm + r]
            row = x_ref[pl.ds(r, 1), :]
            o_ref[pl.ds(b, 1), :] = jnp.maximum(o_ref[pl.ds(b, 1), :], row)
            return 0

        lax.fori_loop(0, bm, step, 0)

        @pl.when(i == grid[0] - 1)
        def _fix():
            v = o_ref[:]
            o_ref[:] = jnp.where(v > -jnp.inf, v, 0.0)

    return pl.pallas_call(
        body,
        grid_spec=pltpu.PrefetchScalarGridSpec(
            num_scalar_prefetch=1,
            grid=grid,
            in_specs=[pl.BlockSpec((bm, C), lambda i, b: (i, 0))],
            out_specs=pl.BlockSpec((nb, C), lambda i, b: (0, 0)),
        ),
        out_shape=jax.ShapeDtypeStruct((nb, C), jnp.float32),
    )(batch, x)


_NC, _NS, _LN = 2, 16, 16
_NTILE = _NC * _NS


def _sc_edge_agg(feat, s1T, s2T, src, dst, H, F, n_pad):
    e = s1T[:, src] + s2T[:, dst]
    e = jnp.where(e > 0, e, 0.2 * e)
    ex = jnp.exp(e)
    contrib = ex[:, :, None] * feat[src][None, :, :]
    z = jax.ops.segment_sum(
        contrib.transpose(1, 0, 2), dst, num_segments=n_pad)
    z = z.transpose(1, 0, 2)
    return jnp.stack([z, jnp.zeros_like(z)])


def _prep_edges(ei, n, n_pad):
    E = ei.shape[1] + n
    EP = ((E + _NTILE * 16 - 1) // (_NTILE * 16)) * (_NTILE * 16)
    loops = jnp.arange(n, dtype=ei.dtype)
    src = jnp.concatenate([ei[0], loops,
                           jnp.full((EP - E,), n, ei.dtype)])
    dst = jnp.concatenate([ei[1], loops,
                           jnp.full((EP - E,), n, ei.dtype)])
    return src.astype(jnp.int32), dst.astype(jnp.int32)


def _pad_scoresT(sT, n, n_pad):
    H = sT.shape[1]
    out = jnp.full((H, n_pad), -1e30, jnp.float32)
    return lax.dynamic_update_slice(out, sT.T, (0, 0))


def _graph_branch(x, ei, batch, nb, P, pfx, fcW, fcB):
    n = x.shape[0]
    n_pad = ((n + 1 + 127) // 128) * 128
    src, dst = _prep_edges(ei, n, n_pad)

    W = P[pfx + 'aW']
    A_s = _head_diag(P[pfx + 'aAs'])
    A_d = _head_diag(P[pfx + 'aAd'])
    _, s1, s2 = _gat_pre(x, W, A_s, A_d, want_h=False, bm=1000)
    featA = jnp.zeros((n_pad, 80), jnp.float32)
    featA = featA.at[:n, :78].set(x).at[:n, 78].set(1.0)
    z4 = _sc_edge_agg(featA, _pad_scoresT(s1, n, n_pad),
                      _pad_scoresT(s2, n, n_pad), src, dst, 10, 80, n_pad)
    Wp = jnp.concatenate([W, jnp.zeros((2, W.shape[1]), jnp.float32)], 0)
    h1 = _gat_postA(z4, Wp, P[pfx + 'aB'], n)

    W2 = P[pfx + 'bW']
    A2s = _head_diag(P[pfx + 'bAs'])
    A2d = _head_diag(P[pfx + 'bAd'])
    hB, t1, t2 = _gat_pre(h1, W2, A2s, A2d, want_h=True, bm=1000)
    featB = jnp.zeros((n_pad, 144), jnp.float32)
    featB = featB.at[:n, :128].set(hB).at[:n, 128].set(1.0)
    zB = _sc_edge_agg(featB, _pad_scoresT(t1, n, n_pad),
                      _pad_scoresT(t2, n, n_pad), src, dst, 1, 144, n_pad)
    h2 = _gat_postB(zB.reshape(2, n_pad, 144), P[pfx + 'bB'], n)

    pooled = _gmp(h2, batch, nb)
    return _mm(pooled, fcW, bias=fcB, act=_relu)


def _mha_tail(allf, P):
    nb = allf.shape[0]
    a2 = allf.reshape(nb * 9, 256)
    q = _mm(a2, P['WQ'], bm=1024).reshape(nb, 9, 4, 64)
    k = _mm(a2, P['WK'], bm=1024).reshape(nb, 9, 4, 64)
    v = _mm(a2, P['WV'], bm=1024).reshape(nb, 9, 4, 64)
    r0 = _mm(a2, P['WR'], bm=1024).reshape(nb, 9, 256)
    att = jax.nn.softmax(
        jnp.einsum('bqhd,bkhd->bhqk', q, k) / (64.0 ** 0.5), axis=-1)
    r = jnp.einsum('bhqk,bkhd->bqhd', att, v).reshape(nb, 9, 256)
    mh = _relu(r + r0).reshape(nb, 2304)

    t = _ln_op(mh, P['lnG'], P['lnB'])
    sqrt2 = math.sqrt(2.0)
    t = _mm(t, P['t1W'], bias=P['t1B'],
            act=lambda z: 0.5 * z * (1.0 + lax.erf(z / sqrt2)))
    t = _mm(t, P['t2W'], bias=P['t2B'])
    return t


def _ln_op(x, g, b):
    M, N = x.shape

    def body(x_ref, g_ref, b_ref, o_ref):
        z = x_ref[:]
        mu = jnp.mean(z, axis=-1, keepdims=True)
        var = jnp.mean((z - mu) ** 2, axis=-1, keepdims=True)
        o_ref[:] = (z - mu) * lax.rsqrt(var + 1e-5) * g_ref[:].reshape(1, N) \
            + b_ref[:].reshape(1, N)

    return pl.pallas_call(
        body,
        grid=(1,),
        in_specs=[
            pl.BlockSpec((M, N), lambda i: (0, 0)),
            pl.BlockSpec((N,), lambda i: (0,)),
            pl.BlockSpec((N,), lambda i: (0,)),
        ],
        out_specs=pl.BlockSpec((M, N), lambda i: (0, 0)),
        out_shape=jax.ShapeDtypeStruct((M, N), jnp.float32),
    )(x, g, b)


def kernel(x1, edge_index1, batch1, x2, edge_index2, batch2, context,
           espf1, mixfp1, espf2, mixfp2, cnv, pw, params):
    P = params
    nb = context.shape[0]

    s1 = _graph_branch(x1, edge_index1, batch1, nb, P, 'g1', P['fc1W'], P['fc1B'])
    s2 = _graph_branch(x2, edge_index2, batch2, nb, P, 'g2', P['fc2W'], P['fc2B'])

    ctx = _mm(context[:, 0, :], P['pcW'], bias=P['pcB'], act=_relu,
              ln=(P['pcG'], P['pcBe']))
    cn = _mm(cnv[:, 0, :], P['pnW'], bias=P['pnB'], act=_relu,
             ln=(P['pnG'], P['pnBe']))
    y = _mm(pw[:, 0, :], P['pwW1'], bias=P['pwB1'], act=_relu,
            ln=(P['pwG1'], P['pwBe1']))
    pwf = _mm(y, P['pwW2'], bias=P['pwB2'], act=_relu,
              ln=(P['pwG2'], P['pwBe2']))
    e1 = _mm(espf1, P['pe1W'], bias=P['pe1B'], act=_relu, ln=(P['pe1G'], P['pe1Be']))
    e2 = _mm(espf2, P['pe2W'], bias=P['pe2B'], act=_relu, ln=(P['pe2G'], P['pe2Be']))
    m1 = _mm(mixfp1, P['pm1W'], bias=P['pm1B'], act=_relu, ln=(P['pm1G'], P['pm1Be']))
    m2 = _mm(mixfp2, P['pm2W'], bias=P['pm2B'], act=_relu, ln=(P['pm2G'], P['pm2Be']))

    allf = jnp.stack([s1, m1, e1, ctx, cn, pwf, s2, m2, e2], axis=1)
    mx = _mha_tail(allf, P) + allf.reshape(nb, 2304)

    af = mx.reshape(nb, 9, 256)
    d1 = af[:, :3, :].reshape(nb, 768)
    d2 = af[:, 6:, :].reshape(nb, 768)
    cell = af[:, 3:6, :].reshape(nb, 768)
    comb = _mm(jnp.concatenate([d1, d2], axis=1), P['combW'])
    ther = comb * cell
    out = _mm(_ln_op(ther, P['trG'], P['trB']),
              jnp.pad(P['tW'], ((0, 0), (0, 127))),
              bias=jnp.pad(P['tB'], (0, 127)))[:, :1]
    return (out, ther)
